# baseline (speedup 1.0000x reference)
import jax
import jax.numpy as jnp
from jax import lax
from jax.experimental import pallas as pl
from jax.experimental.pallas import tpu as pltpu

_NE = 512
_D = 32
_B = 16
_P = 64 * 64
_R = 2048
_NJ = _P // _R
_NS = _B * _NJ
_SCALE = 1.26 / (_B * _P * _D)
_NQ = 4
_EC = _NE // _NQ


def _vq_block(x_ref, w_ref, xsq_ref, wsq_ref,
              quant_ref, loss_ref, disc_ref, s0, s1, sem):
    b = pl.program_id(0)
    j = pl.program_id(1)
    step = b * _NJ + j

    xb = x_ref[0]
    w = w_ref[...]

    xbt = xb.T
    neg2s = jax.lax.dot_general(
        xbt, w * -2.0, (((1,), (1,)), ((), ())),
        preferred_element_type=jnp.float32,
    )
    xsq_col = xsq_ref[0, 0].T
    dist = (xsq_col + wsq_ref[...]) + neg2s

    idx = jnp.argmin(dist, axis=-1)

    eq = jax.lax.broadcasted_iota(jnp.int32, (_NE, _R), 0) == idx[None, :]

    par = step % 2

    @pl.when(par == 0)
    def _st0():
        s0[...] = eq.astype(jnp.int32)

    @pl.when(par == 1)
    def _st1():
        s1[...] = eq.astype(jnp.int32)

    for k in range(_NQ):
        @pl.when(par == 0)
        def _go0():
            pltpu.make_async_copy(
                s0.at[pl.ds(k * _EC, _EC)],
                disc_ref.at[b, pl.ds(k * _EC, _EC), pl.ds(j * _R, _R)],
                sem.at[0, k],
            ).start()

        @pl.when(par == 1)
        def _go1():
            pltpu.make_async_copy(
                s1.at[pl.ds(k * _EC, _EC)],
                disc_ref.at[b, pl.ds(k * _EC, _EC), pl.ds(j * _R, _R)],
                sem.at[1, k],
            ).start()

    ohf = eq.astype(jnp.float32)
    quant = jax.lax.dot_general(
        w, ohf, (((0,), (0,)), ((), ())),
        preferred_element_type=jnp.float32,
    )
    quant_ref[0] = quant

    part = jnp.sum((quant - xb) ** 2)

    @pl.when(step == 0)
    def _init():
        loss_ref[0, 0] = part

    @pl.when(step != 0)
    def _acc():
        loss_ref[0, 0] += part

    @pl.when(step == _NS - 1)
    def _fin():
        loss_ref[0, 0] *= _SCALE

    for k in range(_NQ):
        @pl.when(step >= 1)
        def _wprev():
            pltpu.make_async_copy(
                s0.at[pl.ds(k * _EC, _EC)],
                disc_ref.at[b, pl.ds(k * _EC, _EC), pl.ds(j * _R, _R)],
                sem.at[1 - par, k],
            ).wait()

        @pl.when(step == _NS - 1)
        def _wlast():
            pltpu.make_async_copy(
                s0.at[pl.ds(k * _EC, _EC)],
                disc_ref.at[b, pl.ds(k * _EC, _EC), pl.ds(j * _R, _R)],
                sem.at[par, k],
            ).wait()


@jax.jit
def kernel(x, W):
    xr = x.reshape(_B, _D, _P)
    flat = jnp.moveaxis(x, 1, -1).reshape(-1, _D)
    xsq = jnp.sum(flat ** 2, axis=-1).reshape(_B, _NJ, 1, _R)
    wsq = jnp.sum(W ** 2, axis=-1).reshape(1, _NE)

    quant, loss, disc = pl.pallas_call(
        _vq_block,
        grid=(_B, _NJ),
        in_specs=[
            pl.BlockSpec((1, _D, _R), lambda b, j: (b, 0, j)),
            pl.BlockSpec((_NE, _D), lambda b, j: (0, 0)),
            pl.BlockSpec((1, 1, 1, _R), lambda b, j: (b, j, 0, 0)),
            pl.BlockSpec((1, _NE), lambda b, j: (0, 0)),
        ],
        out_specs=[
            pl.BlockSpec((1, _D, _R), lambda b, j: (b, 0, j)),
            pl.BlockSpec((1, 1), lambda b, j: (0, 0), memory_space=pltpu.SMEM),
            pl.BlockSpec(memory_space=pl.ANY),
        ],
        out_shape=[
            jax.ShapeDtypeStruct((_B, _D, _P), jnp.float32),
            jax.ShapeDtypeStruct((1, 1), jnp.float32),
            jax.ShapeDtypeStruct((_B, _NE, _P), jnp.int32),
        ],
        scratch_shapes=[
            pltpu.VMEM((_NE, _R), jnp.int32),
            pltpu.VMEM((_NE, _R), jnp.int32),
            pltpu.SemaphoreType.DMA((2, _NQ)),
        ],
    )(xr, W, xsq, wsq)
    return (
        quant.reshape(_B, _D, 64, 64),
        loss[0, 0],
        disc.reshape(_B, _NE, 64, 64),
    )


# speedup vs baseline: 1.0695x; 1.0695x over previous
import jax
import jax.numpy as jnp
from jax import lax
from jax.experimental import pallas as pl
from jax.experimental.pallas import tpu as pltpu

_NE = 512
_D = 32
_B = 16
_P = 64 * 64
_R = 4096
_NJ = _P // _R
_NS = _B * _NJ
_SCALE = 1.26 / (_B * _P * _D)
_NQ = 4
_EC = _NE // _NQ


def _vq_block(x_ref, w_ref, xsq_ref, wsq_ref,
              quant_ref, loss_ref, disc_ref, s0, s1, sem):
    b = pl.program_id(0)
    j = pl.program_id(1)
    step = b * _NJ + j

    xb = x_ref[0]
    w = w_ref[...]

    xbt = xb.T
    neg2s = jax.lax.dot_general(
        xbt, w * -2.0, (((1,), (1,)), ((), ())),
        preferred_element_type=jnp.float32,
    )
    xsq_col = xsq_ref[0, 0].T
    dist = (xsq_col + wsq_ref[...]) + neg2s

    idx = jnp.argmin(dist, axis=-1)

    eq = jax.lax.broadcasted_iota(jnp.int32, (_NE, _R), 0) == idx[None, :]

    par = step % 2

    @pl.when(par == 0)
    def _st0():
        s0[...] = eq.astype(jnp.int32)

    @pl.when(par == 1)
    def _st1():
        s1[...] = eq.astype(jnp.int32)

    for k in range(_NQ):
        @pl.when(par == 0)
        def _go0():
            pltpu.make_async_copy(
                s0.at[pl.ds(k * _EC, _EC)],
                disc_ref.at[b, pl.ds(k * _EC, _EC), pl.ds(j * _R, _R)],
                sem.at[0, k],
            ).start()

        @pl.when(par == 1)
        def _go1():
            pltpu.make_async_copy(
                s1.at[pl.ds(k * _EC, _EC)],
                disc_ref.at[b, pl.ds(k * _EC, _EC), pl.ds(j * _R, _R)],
                sem.at[1, k],
            ).start()

    ohf = eq.astype(jnp.float32)
    quant = jax.lax.dot_general(
        w, ohf, (((0,), (0,)), ((), ())),
        preferred_element_type=jnp.float32,
    )
    quant_ref[0] = quant

    part = jnp.sum((quant - xb) ** 2)

    @pl.when(step == 0)
    def _init():
        loss_ref[0, 0] = part

    @pl.when(step != 0)
    def _acc():
        loss_ref[0, 0] += part

    @pl.when(step == _NS - 1)
    def _fin():
        loss_ref[0, 0] *= _SCALE

    for k in range(_NQ):
        @pl.when(step >= 1)
        def _wprev():
            pltpu.make_async_copy(
                s0.at[pl.ds(k * _EC, _EC)],
                disc_ref.at[b, pl.ds(k * _EC, _EC), pl.ds(j * _R, _R)],
                sem.at[1 - par, k],
            ).wait()

        @pl.when(step == _NS - 1)
        def _wlast():
            pltpu.make_async_copy(
                s0.at[pl.ds(k * _EC, _EC)],
                disc_ref.at[b, pl.ds(k * _EC, _EC), pl.ds(j * _R, _R)],
                sem.at[par, k],
            ).wait()


@jax.jit
def kernel(x, W):
    xr = x.reshape(_B, _D, _P)
    flat = jnp.moveaxis(x, 1, -1).reshape(-1, _D)
    xsq = jnp.sum(flat ** 2, axis=-1).reshape(_B, _NJ, 1, _R)
    wsq = jnp.sum(W ** 2, axis=-1).reshape(1, _NE)

    quant, loss, disc = pl.pallas_call(
        _vq_block,
        grid=(_B, _NJ),
        in_specs=[
            pl.BlockSpec((1, _D, _R), lambda b, j: (b, 0, j)),
            pl.BlockSpec((_NE, _D), lambda b, j: (0, 0)),
            pl.BlockSpec((1, 1, 1, _R), lambda b, j: (b, j, 0, 0)),
            pl.BlockSpec((1, _NE), lambda b, j: (0, 0)),
        ],
        out_specs=[
            pl.BlockSpec((1, _D, _R), lambda b, j: (b, 0, j)),
            pl.BlockSpec((1, 1), lambda b, j: (0, 0), memory_space=pltpu.SMEM),
            pl.BlockSpec(memory_space=pl.ANY),
        ],
        out_shape=[
            jax.ShapeDtypeStruct((_B, _D, _P), jnp.float32),
            jax.ShapeDtypeStruct((1, 1), jnp.float32),
            jax.ShapeDtypeStruct((_B, _NE, _P), jnp.int32),
        ],
        scratch_shapes=[
            pltpu.VMEM((_NE, _R), jnp.int32),
            pltpu.VMEM((_NE, _R), jnp.int32),
            pltpu.SemaphoreType.DMA((2, _NQ)),
        ],
    )(xr, W, xsq, wsq)
    return (
        quant.reshape(_B, _D, 64, 64),
        loss[0, 0],
        disc.reshape(_B, _NE, 64, 64),
    )
